# baseline (speedup 1.0000x reference)
import jax
import jax.numpy as jnp
from jax import lax
from jax.experimental import pallas as pl
from jax.experimental.pallas import tpu as pltpu

_B = 512
_N = 1024
_F = 256
_TOPK = 10
_ROWS = 128
_SROWS = 16


def _xor_perm(x, j):
    if j >= 128:
        k = j // 128
        order = [s ^ k for s in range(8)]
        slices = [x[:, s:s + 1, :] for s in order]
        return jnp.concatenate(slices, axis=1)
    return jnp.concatenate([x[..., j:], x[..., :j]], axis=-1)


def _xor_perm_r(x, j):
    return jnp.concatenate([x[..., -j:], x[..., :-j]], axis=-1)


def _sort_block(d_sb, bcs):
    r = d_sb.shape[0]
    key = lax.bitcast_convert_type(d_sb, jnp.int32).reshape(r, 8, 128)
    lane = lax.broadcasted_iota(jnp.int32, (r, 8, 128), 2)
    sgrp = lax.broadcasted_iota(jnp.int32, (r, 8, 128), 1)
    idx = sgrp * 128 + lane

    ksz = 2
    while ksz <= _N:
        j = ksz // 2
        while j >= 1:
            if j >= 64:
                pk = _xor_perm(key, j)
                pi = _xor_perm(idx, j)
            else:
                bc = bcs[j]
                pk = jnp.where(bc, _xor_perm(key, j), _xor_perm_r(key, j))
                pi = jnp.where(bc, _xor_perm(idx, j), _xor_perm_r(idx, j))
            ps = (pk < key) | ((pk == key) & (pi < idx))
            if ksz == _N:
                want_min = bcs[j]
            else:
                want_min = bcs[j] == bcs[ksz]
            take = ps == want_min
            key = jnp.where(take, pk, key)
            idx = jnp.where(take, pi, idx)
            j //= 2
        ksz *= 2
    return idx.reshape(r, _N)


def _ng_kernel(x_ref, ct_ref, d_ref, k_ref, z_ref):
    xb = x_ref[...]
    ct = ct_ref[...]

    xg = [[lax.broadcast_in_dim(
               xb[:, 128 * b + 8 * j:128 * b + 8 * j + 8].reshape(_ROWS, 8, 1),
               (_ROWS, 8, 128), (0, 1, 2))
           for j in range(16)] for b in range(2)]

    chunks = []
    for uc in range(0, _N, 128):
        parts = []
        for b in range(2):
            fb = 128 * b
            acc = None
            for j in range(16):
                cT_tile = ct[fb + 8 * j:fb + 8 * j + 8, uc:uc + 128]
                diff = xg[b][j] - cT_tile[None, :, :]
                sq = diff * diff
                acc = sq if acc is None else acc + sq
            acc = acc[:, 0:4, :] + acc[:, 4:8, :]
            acc = acc[:, 0:2, :] + acc[:, 2:4, :]
            acc = acc[:, 0:1, :] + acc[:, 1:2, :]
            parts.append(acc.reshape(_ROWS, 128))
        chunks.append(parts[0] + parts[1])
    dsq = jnp.concatenate(chunks, axis=1)
    d = jnp.sqrt(dsq)
    d_ref[...] = d

    lane1 = lax.broadcasted_iota(jnp.int32, (1, 8, 128), 2)
    sgrp1 = lax.broadcasted_iota(jnp.int32, (1, 8, 128), 1)
    bcs = {}
    for t in range(10):
        i = 1 << t
        bcs[i] = ((lane1 & i) == 0) if i < 128 else ((sgrp1 & (i // 128)) == 0)

    lane2 = lax.broadcasted_iota(jnp.int32, (_SROWS, _N), 1)
    for r0 in range(0, _ROWS, _SROWS):
        idx = _sort_block(d[r0:r0 + _SROWS], bcs)
        k_ref[r0:r0 + _SROWS, :] = idx
        z = jnp.zeros((_SROWS, _N), jnp.float32)
        for m in range(_TOPK):
            col = lax.slice(idx, (0, m), (_SROWS, m + 1))
            z = z + jnp.where(col == lane2, 1.0, 0.0)
        z_ref[r0:r0 + _SROWS, :] = z


@jax.jit
def kernel(x, c):
    grid = _B // _ROWS
    ct = c.T
    d, k, z = pl.pallas_call(
        _ng_kernel,
        grid=(grid,),
        in_specs=[
            pl.BlockSpec((_ROWS, _F), lambda i: (i, 0)),
            pl.BlockSpec((_F, _N), lambda i: (0, 0)),
        ],
        out_specs=[
            pl.BlockSpec((_ROWS, _N), lambda i: (i, 0)),
            pl.BlockSpec((_ROWS, _N), lambda i: (i, 0)),
            pl.BlockSpec((_ROWS, _N), lambda i: (i, 0)),
        ],
        out_shape=[
            jax.ShapeDtypeStruct((_B, _N), jnp.float32),
            jax.ShapeDtypeStruct((_B, _N), jnp.int32),
            jax.ShapeDtypeStruct((_B, _N), jnp.float32),
        ],
    )(x, ct)
    return (d, k, z)


# speedup vs baseline: 1.1889x; 1.1889x over previous
import jax
import jax.numpy as jnp
from jax import lax
from jax.experimental import pallas as pl
from jax.experimental.pallas import tpu as pltpu

_B = 512
_N = 1024
_F = 256
_TOPK = 10
_ROWS = 32
_SROWS = 16


def _xor_perm(x, j):
    if j >= 128:
        k = j // 128
        order = [s ^ k for s in range(8)]
        slices = [x[:, s:s + 1, :] for s in order]
        return jnp.concatenate(slices, axis=1)
    return jnp.concatenate([x[..., j:], x[..., :j]], axis=-1)


def _xor_perm_r(x, j):
    return jnp.concatenate([x[..., -j:], x[..., :-j]], axis=-1)


def _sort_block(d_sb, bcs):
    r = d_sb.shape[0]
    key = lax.bitcast_convert_type(d_sb, jnp.int32).reshape(r, 8, 128)
    lane = lax.broadcasted_iota(jnp.int32, (r, 8, 128), 2)
    sgrp = lax.broadcasted_iota(jnp.int32, (r, 8, 128), 1)
    idx = sgrp * 128 + lane

    ksz = 2
    while ksz <= _N:
        j = ksz // 2
        while j >= 1:
            if j >= 64:
                pk = _xor_perm(key, j)
                pi = _xor_perm(idx, j)
            else:
                bc = bcs[j]
                pk = jnp.where(bc, _xor_perm(key, j), _xor_perm_r(key, j))
                pi = jnp.where(bc, _xor_perm(idx, j), _xor_perm_r(idx, j))
            ps = (pk < key) | ((pk == key) & (pi < idx))
            if ksz == _N:
                want_min = bcs[j]
            else:
                want_min = bcs[j] == bcs[ksz]
            take = ps == want_min
            key = jnp.where(take, pk, key)
            idx = jnp.where(take, pi, idx)
            j //= 2
        ksz *= 2
    return idx.reshape(r, _N)


def _ng_kernel(x_ref, ct_ref, d_ref, k_ref, z_ref):
    xb = x_ref[...]
    ct = ct_ref[...]

    xg = [[lax.broadcast_in_dim(
               xb[:, 128 * b + 8 * j:128 * b + 8 * j + 8].reshape(_ROWS, 8, 1),
               (_ROWS, 8, 128), (0, 1, 2))
           for j in range(16)] for b in range(2)]

    chunks = []
    for uc in range(0, _N, 128):
        parts = []
        for b in range(2):
            fb = 128 * b
            acc = None
            for j in range(16):
                cT_tile = ct[fb + 8 * j:fb + 8 * j + 8, uc:uc + 128]
                diff = xg[b][j] - cT_tile[None, :, :]
                sq = diff * diff
                acc = sq if acc is None else acc + sq
            acc = acc[:, 0:4, :] + acc[:, 4:8, :]
            acc = acc[:, 0:2, :] + acc[:, 2:4, :]
            acc = acc[:, 0:1, :] + acc[:, 1:2, :]
            parts.append(acc.reshape(_ROWS, 128))
        chunks.append(parts[0] + parts[1])
    dsq = jnp.concatenate(chunks, axis=1)
    d = jnp.sqrt(dsq)
    d_ref[...] = d

    lane1 = lax.broadcasted_iota(jnp.int32, (1, 8, 128), 2)
    sgrp1 = lax.broadcasted_iota(jnp.int32, (1, 8, 128), 1)
    bcs = {}
    for t in range(10):
        i = 1 << t
        bcs[i] = ((lane1 & i) == 0) if i < 128 else ((sgrp1 & (i // 128)) == 0)

    lane2 = lax.broadcasted_iota(jnp.int32, (_SROWS, _N), 1)
    for r0 in range(0, _ROWS, _SROWS):
        idx = _sort_block(d[r0:r0 + _SROWS], bcs)
        k_ref[r0:r0 + _SROWS, :] = idx
        z = jnp.zeros((_SROWS, _N), jnp.float32)
        for m in range(_TOPK):
            col = lax.slice(idx, (0, m), (_SROWS, m + 1))
            z = z + jnp.where(col == lane2, 1.0, 0.0)
        z_ref[r0:r0 + _SROWS, :] = z


@jax.jit
def kernel(x, c):
    grid = _B // _ROWS
    ct = c.T
    d, k, z = pl.pallas_call(
        _ng_kernel,
        grid=(grid,),
        in_specs=[
            pl.BlockSpec((_ROWS, _F), lambda i: (i, 0)),
            pl.BlockSpec((_F, _N), lambda i: (0, 0)),
        ],
        out_specs=[
            pl.BlockSpec((_ROWS, _N), lambda i: (i, 0)),
            pl.BlockSpec((_ROWS, _N), lambda i: (i, 0)),
            pl.BlockSpec((_ROWS, _N), lambda i: (i, 0)),
        ],
        out_shape=[
            jax.ShapeDtypeStruct((_B, _N), jnp.float32),
            jax.ShapeDtypeStruct((_B, _N), jnp.int32),
            jax.ShapeDtypeStruct((_B, _N), jnp.float32),
        ],
    )(x, ct)
    return (d, k, z)


# speedup vs baseline: 1.4604x; 1.2283x over previous
import jax
import jax.numpy as jnp
from jax import lax
from jax.experimental import pallas as pl
from jax.experimental.pallas import tpu as pltpu

_B = 512
_N = 1024
_F = 256
_TOPK = 10
_ROWS = 64
_SROWS = 32


def _xor_perm(x, j):
    if j >= 128:
        k = j // 128
        order = [s ^ k for s in range(8)]
        slices = [x[:, s:s + 1, :] for s in order]
        return jnp.concatenate(slices, axis=1)
    return jnp.concatenate([x[..., j:], x[..., :j]], axis=-1)


def _xor_perm_r(x, j):
    return jnp.concatenate([x[..., -j:], x[..., :-j]], axis=-1)


def _sort_block(d_sb, bcs):
    r = d_sb.shape[0]
    key = lax.bitcast_convert_type(d_sb, jnp.int32).reshape(r, 8, 128)
    lane = lax.broadcasted_iota(jnp.int32, (r, 8, 128), 2)
    sgrp = lax.broadcasted_iota(jnp.int32, (r, 8, 128), 1)
    idx = sgrp * 128 + lane

    ksz = 2
    while ksz <= _N:
        j = ksz // 2
        while j >= 1:
            if j >= 64:
                pk = _xor_perm(key, j)
                pi = _xor_perm(idx, j)
            else:
                bc = bcs[j]
                pk = jnp.where(bc, _xor_perm(key, j), _xor_perm_r(key, j))
                pi = jnp.where(bc, _xor_perm(idx, j), _xor_perm_r(idx, j))
            ps = (pk < key) | ((pk == key) & (pi < idx))
            if ksz == _N:
                want_min = bcs[j]
            else:
                want_min = bcs[j] == bcs[ksz]
            take = ps == want_min
            key = jnp.where(take, pk, key)
            idx = jnp.where(take, pi, idx)
            j //= 2
        ksz *= 2
    return idx.reshape(r, _N)


def _ng_kernel(x_ref, ct_ref, d_ref, k_ref, z_ref):
    xb = x_ref[...]
    ct = ct_ref[...]

    xg = [[lax.broadcast_in_dim(
               xb[:, 128 * b + 8 * j:128 * b + 8 * j + 8].reshape(_ROWS, 8, 1),
               (_ROWS, 8, 128), (0, 1, 2))
           for j in range(16)] for b in range(2)]

    chunks = []
    for uc in range(0, _N, 128):
        parts = []
        for b in range(2):
            fb = 128 * b
            acc = None
            for j in range(16):
                cT_tile = ct[fb + 8 * j:fb + 8 * j + 8, uc:uc + 128]
                diff = xg[b][j] - cT_tile[None, :, :]
                sq = diff * diff
                acc = sq if acc is None else acc + sq
            acc = acc[:, 0:4, :] + acc[:, 4:8, :]
            acc = acc[:, 0:2, :] + acc[:, 2:4, :]
            acc = acc[:, 0:1, :] + acc[:, 1:2, :]
            parts.append(acc.reshape(_ROWS, 128))
        chunks.append(parts[0] + parts[1])
    dsq = jnp.concatenate(chunks, axis=1)
    d = jnp.sqrt(dsq)
    d_ref[...] = d

    lane1 = lax.broadcasted_iota(jnp.int32, (1, 8, 128), 2)
    sgrp1 = lax.broadcasted_iota(jnp.int32, (1, 8, 128), 1)
    bcs = {}
    for t in range(10):
        i = 1 << t
        bcs[i] = ((lane1 & i) == 0) if i < 128 else ((sgrp1 & (i // 128)) == 0)

    lane2 = lax.broadcasted_iota(jnp.int32, (_SROWS, _N), 1)
    for r0 in range(0, _ROWS, _SROWS):
        idx = _sort_block(d[r0:r0 + _SROWS], bcs)
        k_ref[r0:r0 + _SROWS, :] = idx
        z = jnp.zeros((_SROWS, _N), jnp.float32)
        for m in range(_TOPK):
            col = lax.slice(idx, (0, m), (_SROWS, m + 1))
            z = z + jnp.where(col == lane2, 1.0, 0.0)
        z_ref[r0:r0 + _SROWS, :] = z


@jax.jit
def kernel(x, c):
    grid = _B // _ROWS
    ct = c.T
    d, k, z = pl.pallas_call(
        _ng_kernel,
        grid=(grid,),
        in_specs=[
            pl.BlockSpec((_ROWS, _F), lambda i: (i, 0)),
            pl.BlockSpec((_F, _N), lambda i: (0, 0)),
        ],
        out_specs=[
            pl.BlockSpec((_ROWS, _N), lambda i: (i, 0)),
            pl.BlockSpec((_ROWS, _N), lambda i: (i, 0)),
            pl.BlockSpec((_ROWS, _N), lambda i: (i, 0)),
        ],
        out_shape=[
            jax.ShapeDtypeStruct((_B, _N), jnp.float32),
            jax.ShapeDtypeStruct((_B, _N), jnp.int32),
            jax.ShapeDtypeStruct((_B, _N), jnp.float32),
        ],
    )(x, ct)
    return (d, k, z)
